# Initial kernel scaffold; baseline (speedup 1.0000x reference)
import jax
import jax.numpy as jnp
from jax.experimental import pallas as pl


def kernel(in_feat, edge_index, W1, b1, W2, b2):
    raise NotImplementedError("write your pallas kernel here")


# speedup vs baseline: 7.1871x; 7.1871x over previous
import functools

import jax
import jax.numpy as jnp
from jax import lax
from jax.experimental import pallas as pl
from jax.experimental.pallas import tpu as pltpu
from jax.experimental.pallas import tpu_sc as plsc

N_NODES = 10000
N_EDGES = 160000
NC = 2
NS = 16
CHUNK = 80
EPT = N_EDGES // NS
NCHUNK = EPT // CHUNK
N_PAD = 10240
RPT = N_PAD // NS
DEG_RPT = 640

_mesh = lambda: plsc.VectorSubcoreMesh(
    core_axis_name="c", subcore_axis_name="s", num_cores=NC, num_subcores=NS)


DEG_STEPS = N_EDGES // NS // 16
HROWS = 128
HRPT = HROWS // NS


def _deg_body(ei_ref, deg_ref, idx_v, acc2d, tmp2d, fin2d, sh):
    c = lax.axis_index("c")
    s = lax.axis_index("s")
    ones = jnp.ones((16,), jnp.float32)
    zeros = jnp.zeros((16,), jnp.float32)

    def zero_row(k, carry):
        for j in range(8):
            acc2d[k, pl.ds(j * 16, 16)] = zeros
        return carry

    lax.fori_loop(0, HROWS, zero_row, 0)
    pltpu.sync_copy(ei_ref.at[c, s], idx_v)

    def step(k, carry):
        iv = idx_v[k, :]
        plsc.addupdate_scatter(
            acc2d, [lax.shift_right_logical(iv, 7),
                    lax.bitwise_and(iv, 127)], ones)
        return carry

    lax.fori_loop(0, DEG_STEPS, step, 0)
    pltpu.sync_copy(acc2d, sh.at[s])
    plsc.subcore_barrier()

    for r in range(HRPT):
        for j in range(8):
            fin2d[r, pl.ds(j * 16, 16)] = zeros

    def red(j, carry):
        pltpu.sync_copy(sh.at[j, pl.ds(s * HRPT, HRPT), :], tmp2d)
        for r in range(HRPT):
            for q in range(8):
                sl = pl.ds(q * 16, 16)
                fin2d[r, sl] = fin2d[r, sl] + tmp2d[r, sl]
        return carry

    lax.fori_loop(0, NS, red, 0)
    pltpu.sync_copy(fin2d, deg_ref.at[c, pl.ds(s * HRPT, HRPT), :])


_deg_call = functools.partial(
    pl.kernel,
    out_type=jax.ShapeDtypeStruct((NC, HROWS, 128), jnp.float32),
    mesh=_mesh(),
    scratch_types=[
        pltpu.VMEM((DEG_STEPS, 16), jnp.int32),
        pltpu.VMEM((HROWS, 128), jnp.float32),
        pltpu.VMEM((HRPT, 128), jnp.float32),
        pltpu.VMEM((HRPT, 128), jnp.float32),
        pltpu.VMEM_SHARED((NS, HROWS, 128), jnp.float32),
    ],
    compiler_params=pltpu.CompilerParams(needs_layout_passes=False),
)(_deg_body)


def _seg_body(dh, h_ref, ei_ref, zeros_hbm, agg_ref,
              idx_s, idx_d, rows, acc, sem):
    c = lax.axis_index("c")
    s = lax.axis_index("s")
    r0 = s * RPT
    pltpu.sync_copy(zeros_hbm, acc.at[pl.ds(r0, RPT), :])
    pltpu.sync_copy(ei_ref.at[0, s], idx_s)
    pltpu.sync_copy(ei_ref.at[1, s], idx_d)
    off = c * N_NODES

    def add_off(r, carry):
        for i in range(CHUNK // 16):
            sl = pl.ds(i * 16, 16)
            idx_s[r, sl] = idx_s[r, sl] + off
        return carry

    lax.fori_loop(0, NCHUNK, add_off, 0)
    plsc.subcore_barrier()

    def step(k, carry):
        pltpu.async_copy(h_ref.at[idx_s.at[k]], rows, sem).wait()
        pltpu.sync_copy(rows, acc.at[idx_d.at[k]], add=True)
        return carry

    lax.fori_loop(0, NCHUNK, step, 0)
    plsc.subcore_barrier()
    pltpu.sync_copy(acc.at[pl.ds(r0, RPT), :],
                    agg_ref.at[c, pl.ds(r0, RPT), :])


def _make_seg(dh):
    return functools.partial(
        pl.kernel,
        out_type=jax.ShapeDtypeStruct((NC, N_PAD, dh), jnp.float32),
        mesh=_mesh(),
        scratch_types=[
            pltpu.VMEM((NCHUNK, CHUNK), jnp.int32),
            pltpu.VMEM((NCHUNK, CHUNK), jnp.int32),
            pltpu.VMEM((CHUNK, dh), jnp.float32),
            pltpu.VMEM_SHARED((N_PAD, dh), jnp.float32),
            pltpu.SemaphoreType.DMA,
        ],
    )(functools.partial(_seg_body, dh))


_seg128 = _make_seg(128)


E_CHUNK2 = 40
E_SLABS2 = 32
NCHUNK2 = N_EDGES // (E_SLABS2 * E_CHUNK2)


def _seg2_body(h_ref, ei_ref, zeros_hbm, agg_ref, idx_s, idx_d, rows, acc, sem):
    c = lax.axis_index("c")
    s = lax.axis_index("s")
    w = c * NS + s
    r0 = s * RPT
    pltpu.sync_copy(zeros_hbm, acc.at[pl.ds(r0, RPT), :])
    pltpu.sync_copy(ei_ref.at[0, w], idx_s)
    pltpu.sync_copy(ei_ref.at[1, w], idx_d)
    plsc.subcore_barrier()

    def step(k, carry):
        pltpu.async_copy(h_ref.at[idx_s.at[k]], rows, sem).wait()
        pltpu.sync_copy(rows, acc.at[idx_d.at[k]], add=True)
        return carry

    lax.fori_loop(0, NCHUNK2, step, 0)
    plsc.subcore_barrier()
    pltpu.sync_copy(acc.at[pl.ds(r0, RPT), :],
                    agg_ref.at[c, pl.ds(r0, RPT), :])


_seg2 = functools.partial(
    pl.kernel,
    out_type=jax.ShapeDtypeStruct((NC, N_PAD, 128), jnp.float32),
    mesh=_mesh(),
    scratch_types=[
        pltpu.VMEM((NCHUNK2, E_CHUNK2), jnp.int32),
        pltpu.VMEM((NCHUNK2, E_CHUNK2), jnp.int32),
        pltpu.VMEM((E_CHUNK2, 128), jnp.float32),
        pltpu.VMEM_SHARED((N_PAD, 128), jnp.float32),
        pltpu.SemaphoreType.DMA,
    ],
)(_seg2_body)


_RB = 400


def _norm(deg_blk):
    return lax.rsqrt(jnp.maximum(deg_blk[:, 0:1], 1.0))


def _mm1_body(x_ref, w_ref, deg_ref, out_ref):
    out_ref[0] = jnp.dot(x_ref[...], w_ref[...],
                         preferred_element_type=jnp.float32) * _norm(deg_ref)


def _mm1(x, w1, deg_out):
    f = x.shape[1]
    return pl.pallas_call(
        _mm1_body,
        grid=(N_NODES // _RB, 2),
        in_specs=[
            pl.BlockSpec((_RB, f), lambda i, j: (i, 0)),
            pl.BlockSpec((f, 128), lambda i, j: (0, j)),
            pl.BlockSpec((_RB, 16), lambda i, j: (i, 0)),
        ],
        out_specs=pl.BlockSpec((1, _RB, 128), lambda i, j: (j, i, 0)),
        out_shape=jax.ShapeDtypeStruct((2, N_NODES, 128), jnp.float32),
    )(x, w1, deg_out)


def _mm2_body(a_ref, w_ref, din_ref, dout_ref, b1_ref, out_ref):
    nd = _norm(din_ref)
    x0 = jnp.maximum(a_ref[0] * nd + b1_ref[0], 0.0)
    x1 = jnp.maximum(a_ref[1] * nd + b1_ref[1], 0.0)
    w = w_ref[...]
    h = (jnp.dot(x0, w[:128], preferred_element_type=jnp.float32)
         + jnp.dot(x1, w[128:], preferred_element_type=jnp.float32))
    out_ref[...] = h * _norm(dout_ref)


def _mm2(agg1, w2, deg_in, deg_out, b1):
    return pl.pallas_call(
        _mm2_body,
        grid=(N_NODES // _RB,),
        in_specs=[
            pl.BlockSpec((2, _RB, 128), lambda i: (0, i, 0)),
            pl.BlockSpec((256, 128), lambda i: (0, 0)),
            pl.BlockSpec((_RB, 16), lambda i: (i, 0)),
            pl.BlockSpec((_RB, 16), lambda i: (i, 0)),
            pl.BlockSpec((2, 128), lambda i: (0, 0)),
        ],
        out_specs=pl.BlockSpec((_RB, 128), lambda i: (i, 0)),
        out_shape=jax.ShapeDtypeStruct((N_NODES, 128), jnp.float32),
    )(agg1, w2, deg_in, deg_out, b1)


def _epi_body(a_ref, din_ref, b2_ref, out_ref):
    nd = _norm(din_ref)
    out_ref[...] = (a_ref[0] + a_ref[1]) * nd + b2_ref[...]


def _epi(agg2, deg_in, b2):
    return pl.pallas_call(
        _epi_body,
        grid=(N_NODES // _RB,),
        in_specs=[
            pl.BlockSpec((2, _RB, 128), lambda i: (0, i, 0)),
            pl.BlockSpec((_RB, 16), lambda i: (i, 0)),
            pl.BlockSpec((1, 128), lambda i: (0, 0)),
        ],
        out_specs=pl.BlockSpec((_RB, 128), lambda i: (i, 0)),
        out_shape=jax.ShapeDtypeStruct((N_NODES, 128), jnp.float32),
    )(agg2, deg_in, b2)


def kernel(in_feat, edge_index, W1, b1, W2, b2):
    ei4 = edge_index.reshape(2, NS, NCHUNK, CHUNK)
    ei32 = edge_index.reshape(2, E_SLABS2, NCHUNK2, E_CHUNK2)
    ei16 = edge_index.reshape(2, NS, DEG_STEPS, 16)
    zer128 = jnp.zeros((RPT, 128), jnp.float32)

    degs = _deg_call(ei16).reshape(2, HROWS * 128)[:, :N_PAD]
    deg_out = jnp.broadcast_to(degs[0][:, None], (N_PAD, 16))
    deg_in = jnp.broadcast_to(degs[1][:, None], (N_PAD, 16))

    h1 = _mm1(in_feat, W1, deg_out)
    agg1 = _seg128(h1.reshape(2 * N_NODES, 128), ei4, zer128)
    h2 = _mm2(agg1, W2, deg_in, deg_out, b1.reshape(2, 128))
    agg2 = _seg2(h2, ei32, zer128)
    return _epi(agg2, deg_in, b2.reshape(1, 128))
